# baseline (speedup 1.0000x reference)
import functools

import jax
import jax.numpy as jnp
from jax import lax
from jax.experimental import pallas as pl
from jax.experimental.pallas import tpu as pltpu
from jax.experimental.pallas import tpu_sc as plsc

_B = 4096
_D = 128
_V = 100000


def _sc_gather(target_word, emb_table):
    info = plsc.get_sparse_core_info()
    nc, ns = info.num_cores, info.num_subcores
    nw = nc * ns
    b_per_w = _B // nw
    mesh = plsc.VectorSubcoreMesh(core_axis_name="c", subcore_axis_name="s")

    @functools.partial(
        pl.kernel,
        mesh=mesh,
        out_type=jax.ShapeDtypeStruct((_B, _D), jnp.float32),
        scratch_types=[
            pltpu.VMEM((b_per_w,), jnp.int32),
            pltpu.VMEM((b_per_w, _D), jnp.float32),
            pltpu.SemaphoreType.DMA,
        ],
    )
    def gather_kernel(idx_hbm, table_hbm, out_hbm, idx_v, rows_v, sem):
        wid = lax.axis_index("s") * nc + lax.axis_index("c")
        base = wid * b_per_w
        pltpu.sync_copy(idx_hbm.at[pl.ds(base, b_per_w)], idx_v)
        pltpu.async_copy(table_hbm.at[idx_v], rows_v, sem).wait()
        pltpu.sync_copy(rows_v, out_hbm.at[pl.ds(base, b_per_w)])

    return gather_kernel(target_word, emb_table)


_BM = 1024
_MT = _B // _BM
_BN = 4096
_NTF = _V // _BN
_TAILW = _V - _NTF * _BN
_RING = 2


_CHUNK = 1024


def _accum_into(emb_ref, w_ref, b_ref, dst_ref, widths):
    off = 0
    for wdt in widths:
        dst_ref[:, pl.ds(off, wdt)] = lax.dot_general(
            emb_ref[...], w_ref[pl.ds(off, wdt), :],
            dimension_numbers=(((1,), (1,)), ((), ())),
            preferred_element_type=jnp.float32,
        ) + b_ref[:, pl.ds(off, wdt)]
        off += wdt


def _mm_kernel(emb_ref, w_ref, b_ref, out_hbm, acc, tailbuf, sems, tsem):
    i = pl.program_id(0)
    j = pl.program_id(1)
    f_ord = i * _NTF + j
    slot = lax.rem(f_ord, _RING)

    @pl.when(jnp.logical_and(j < _NTF, f_ord >= _RING))
    def _wait_prev_full():
        pltpu.make_async_copy(
            acc.at[slot],
            out_hbm.at[pl.ds(0, _BM), pl.ds(0, _BN)],
            sems.at[slot],
        ).wait()

    @pl.when(j < _NTF)
    def _full_tile():
        _accum_into(emb_ref, w_ref, b_ref, acc.at[slot],
                    [_CHUNK] * (_BN // _CHUNK))
        pltpu.make_async_copy(
            acc.at[slot],
            out_hbm.at[pl.ds(i * _BM, _BM), pl.ds(j * _BN, _BN)],
            sems.at[slot],
        ).start()

    @pl.when(j == _NTF)
    def _tail_tile():
        @pl.when(i >= 1)
        def _wait_prev_tail():
            pltpu.make_async_copy(
                tailbuf,
                out_hbm.at[pl.ds(0, _BM), pl.ds(_NTF * _BN, _TAILW)],
                tsem,
            ).wait()

        _accum_into(emb_ref, w_ref, b_ref, tailbuf,
                    [_CHUNK, _TAILW - _CHUNK])
        pltpu.make_async_copy(
            tailbuf,
            out_hbm.at[pl.ds(i * _BM, _BM), pl.ds(_NTF * _BN, _TAILW)],
            tsem,
        ).start()

    @pl.when(jnp.logical_and(i == _MT - 1, j == _NTF))
    def _drain():
        for s in range(_RING):
            pltpu.make_async_copy(
                acc.at[s],
                out_hbm.at[pl.ds(0, _BM), pl.ds(0, _BN)],
                sems.at[s],
            ).wait()
        pltpu.make_async_copy(
            tailbuf,
            out_hbm.at[pl.ds(0, _BM), pl.ds(_NTF * _BN, _TAILW)],
            tsem,
        ).wait()


def _tc_project(embedded, W, b2d):
    return pl.pallas_call(
        _mm_kernel,
        grid=(_MT, _NTF + 1),
        in_specs=[
            pl.BlockSpec((_BM, _D), lambda i, j: (i, 0)),
            pl.BlockSpec((_BN, _D), lambda i, j: (j, 0)),
            pl.BlockSpec((1, _BN), lambda i, j: (0, j)),
        ],
        out_specs=pl.BlockSpec(memory_space=pl.ANY),
        out_shape=jax.ShapeDtypeStruct((_B, _V), jnp.float32),
        scratch_shapes=[
            pltpu.VMEM((_RING, _BM, _BN), jnp.float32),
            pltpu.VMEM((_BM, _TAILW), jnp.float32),
            pltpu.SemaphoreType.DMA((_RING,)),
            pltpu.SemaphoreType.DMA,
        ],
    )(embedded, W, b2d)


def kernel(target_word, emb_table, W, b):
    embedded = _sc_gather(target_word.astype(jnp.int32), emb_table)
    return _tc_project(embedded, W, b.reshape(1, _V))
